# baseline (speedup 1.0000x reference)
import functools

import jax
import jax.numpy as jnp
from jax import lax
from jax.experimental import pallas as pl
from jax.experimental.pallas import tpu as pltpu
from jax.experimental.pallas import tpu_sc as plsc

VOCAB = 100000
EMB_DIM = 128
NUM_CLASSES = 64


_ROWS_PER_BLOCK = 2000


def _project_body(emb_ref, w_ref, b_ref, out_ref):
    out_ref[...] = (
        jnp.dot(emb_ref[...], w_ref[...], preferred_element_type=jnp.float32)
        + b_ref[...]
    )


def _project(emb2, w_blk, b2):
    n_blocks = (VOCAB // 2) // _ROWS_PER_BLOCK
    return pl.pallas_call(
        _project_body,
        grid=(n_blocks,),
        in_specs=[
            pl.BlockSpec((_ROWS_PER_BLOCK, 2 * EMB_DIM), lambda i: (i, 0)),
            pl.BlockSpec((2 * EMB_DIM, 2 * NUM_CLASSES), lambda i: (0, 0)),
            pl.BlockSpec((1, 2 * NUM_CLASSES), lambda i: (0, 0)),
        ],
        out_specs=pl.BlockSpec(
            (_ROWS_PER_BLOCK, 2 * NUM_CLASSES), lambda i: (i, 0)
        ),
        out_shape=jax.ShapeDtypeStruct(
            (VOCAB // 2, 2 * NUM_CLASSES), jnp.float32
        ),
    )(emb2, w_blk, b2)


_CB = 128


def _make_gather(B, L, nw):
    n_chunks_total = (B // _CB) * L
    cpw = n_chunks_total // nw
    assert cpw % 2 == 0
    nbc = B // _CB
    mesh = plsc.VectorSubcoreMesh(core_axis_name="c", subcore_axis_name="s")
    nc = mesh.num_cores

    @functools.partial(
        pl.kernel,
        mesh=mesh,
        out_type=jax.ShapeDtypeStruct((L, NUM_CLASSES, B), jnp.float32),
        scratch_types=[
            pltpu.VMEM((cpw, _CB), jnp.int32),
            pltpu.VMEM((cpw, _CB), jnp.int32),
            pltpu.VMEM((_CB, 2 * NUM_CLASSES), jnp.float32),
            pltpu.VMEM((_CB, 2 * NUM_CLASSES), jnp.float32),
            pltpu.VMEM((NUM_CLASSES, _CB), jnp.float32),
            pltpu.VMEM((NUM_CLASSES, _CB), jnp.float32),
            pltpu.SemaphoreType.DMA,
            pltpu.SemaphoreType.DMA,
            pltpu.SemaphoreType.DMA,
            pltpu.SemaphoreType.DMA,
        ],
        compiler_params=pltpu.CompilerParams(
            use_tc_tiling_on_sc=False, needs_layout_passes=False
        ),
    )
    def gather_k(
        ids_hbm, p_hbm, out_hbm,
        idx_v, hlf_v, gbuf0, gbuf1, obuf0, obuf1, gsem0, gsem1, ssem0, ssem1,
    ):
        wid = lax.axis_index("s") * nc + lax.axis_index("c")
        gbase = wid * cpw
        pltpu.sync_copy(ids_hbm.at[pl.ds(gbase, cpw)], idx_v)

        def hprep(jj, c):
            for k in range(_CB // 16):
                hlf_v[jj, pl.ds(16 * k, 16)] = (
                    idx_v[jj, pl.ds(16 * k, 16)] >> 1
                )
            return c

        lax.fori_loop(0, cpw, hprep, 0, unroll=4)

        pltpu.async_copy(p_hbm.at[hlf_v.at[0]], gbuf0, gsem0)
        pltpu.async_copy(p_hbm.at[hlf_v.at[1]], gbuf1, gsem1)

        iota16 = lax.iota(jnp.int32, 16)

        def compact(j, gbuf, obuf):
            def kstep(k, c):
                rows = iota16 + 16 * k
                par = (idx_v[j, pl.ds(16 * k, 16)] & 1) * NUM_CLASSES
                for cc in range(NUM_CLASSES):
                    obuf[cc, pl.ds(16 * k, 16)] = plsc.load_gather(
                        gbuf, [rows, par + cc]
                    )
                return c

            lax.fori_loop(0, _CB // 16, kstep, 0)

        def out_slice(j):
            g = gbase + j
            return out_hbm.at[g // nbc, :, pl.ds((g % nbc) * _CB, _CB)]

        def half_step(i, j, gbuf, obuf, gsem, ssem):
            pltpu.make_async_copy(p_hbm.at[hlf_v.at[j]], gbuf, gsem).wait()

            @pl.when(i > 0)
            def _():
                pltpu.make_async_copy(obuf, out_slice(j - 2), ssem).wait()

            compact(j, gbuf, obuf)

            @pl.when(j + 2 < cpw)
            def _():
                pltpu.async_copy(p_hbm.at[hlf_v.at[j + 2]], gbuf, gsem)

            pltpu.async_copy(obuf, out_slice(j), ssem)

        def body(i, carry):
            j = 2 * i
            half_step(i, j, gbuf0, obuf0, gsem0, ssem0)
            half_step(i, j + 1, gbuf1, obuf1, gsem1, ssem1)
            return carry

        lax.fori_loop(0, cpw // 2, body, 0)
        pltpu.make_async_copy(obuf0, out_slice(cpw - 2), ssem0).wait()
        pltpu.make_async_copy(obuf1, out_slice(cpw - 1), ssem1).wait()

    return gather_k


def kernel(token_ids, emb_table, W, b):
    B, L = token_ids.shape
    info = plsc.get_sparse_core_info()
    nw = info.num_cores * info.num_subcores

    emb2 = emb_table.reshape(VOCAB // 2, 2 * EMB_DIM)
    w_blk = (
        jnp.zeros((2 * EMB_DIM, 2 * NUM_CLASSES), jnp.float32)
        .at[:EMB_DIM, :NUM_CLASSES].set(W)
        .at[EMB_DIM:, NUM_CLASSES:].set(W)
    )
    b2 = jnp.concatenate([b, b]).reshape(1, 2 * NUM_CLASSES)
    proj = _project(emb2, w_blk, b2)

    ids2 = token_ids.T.reshape((B // _CB) * L, _CB).astype(jnp.int32)
    out3 = _make_gather(B, L, nw)(ids2, proj)
    return out3.transpose(2, 0, 1)


# speedup vs baseline: 1.4306x; 1.4306x over previous
import functools

import jax
import jax.numpy as jnp
from jax import lax
from jax.experimental import pallas as pl
from jax.experimental.pallas import tpu as pltpu
from jax.experimental.pallas import tpu_sc as plsc

VOCAB = 100000
EMB_DIM = 128
NUM_CLASSES = 64


_ROWS_PER_BLOCK = 2000


def _project_body(emb_ref, w_ref, b_ref, out_ref):
    out_ref[...] = (
        jnp.dot(emb_ref[...], w_ref[...], preferred_element_type=jnp.float32)
        + b_ref[...]
    )


def _project(emb2, w_blk, b2):
    n_blocks = (VOCAB // 2) // _ROWS_PER_BLOCK
    return pl.pallas_call(
        _project_body,
        grid=(n_blocks,),
        in_specs=[
            pl.BlockSpec((_ROWS_PER_BLOCK, 2 * EMB_DIM), lambda i: (i, 0)),
            pl.BlockSpec((2 * EMB_DIM, 2 * NUM_CLASSES), lambda i: (0, 0)),
            pl.BlockSpec((1, 2 * NUM_CLASSES), lambda i: (0, 0)),
        ],
        out_specs=pl.BlockSpec(
            (_ROWS_PER_BLOCK, 2 * NUM_CLASSES), lambda i: (i, 0)
        ),
        out_shape=jax.ShapeDtypeStruct(
            (VOCAB // 2, 2 * NUM_CLASSES), jnp.float32
        ),
    )(emb2, w_blk, b2)


_CB = 128


def _make_gather(B, L, nw):
    n_chunks_total = (B // _CB) * L
    cpw = n_chunks_total // nw
    assert cpw % 2 == 0
    nbc = B // _CB
    mesh = plsc.VectorSubcoreMesh(core_axis_name="c", subcore_axis_name="s")
    nc = mesh.num_cores

    @functools.partial(
        pl.kernel,
        mesh=mesh,
        out_type=jax.ShapeDtypeStruct((L, NUM_CLASSES, B), jnp.float32),
        scratch_types=[
            pltpu.VMEM((cpw, _CB), jnp.int32),
            pltpu.VMEM((cpw, _CB), jnp.int32),
            pltpu.VMEM((_CB, 2 * NUM_CLASSES), jnp.float32),
            pltpu.VMEM((_CB, 2 * NUM_CLASSES), jnp.float32),
            pltpu.VMEM((NUM_CLASSES, _CB), jnp.float32),
            pltpu.VMEM((NUM_CLASSES, _CB), jnp.float32),
            pltpu.SemaphoreType.DMA,
            pltpu.SemaphoreType.DMA,
            pltpu.SemaphoreType.DMA,
            pltpu.SemaphoreType.DMA,
        ],
        compiler_params=pltpu.CompilerParams(
            use_tc_tiling_on_sc=False, needs_layout_passes=False
        ),
    )
    def gather_k(
        ids_hbm, p_hbm, out_hbm,
        idx_v, hlf_v, gbuf0, gbuf1, obuf0, obuf1, gsem0, gsem1, ssem0, ssem1,
    ):
        wid = lax.axis_index("s") * nc + lax.axis_index("c")
        gbase = wid * cpw
        pltpu.sync_copy(ids_hbm.at[pl.ds(gbase, cpw)], idx_v)

        def hprep(jj, c):
            for k in range(_CB // 16):
                hlf_v[jj, pl.ds(16 * k, 16)] = (
                    idx_v[jj, pl.ds(16 * k, 16)] >> 1
                )
            return c

        lax.fori_loop(0, cpw, hprep, 0, unroll=4)

        pltpu.async_copy(p_hbm.at[hlf_v.at[0]], gbuf0, gsem0)
        pltpu.async_copy(p_hbm.at[hlf_v.at[1]], gbuf1, gsem1)

        iota16 = lax.iota(jnp.int32, 16)

        def compact(j, gbuf, obuf):
            for k in range(_CB // 16):
                rows = iota16 + 16 * k
                par = (idx_v[j, pl.ds(16 * k, 16)] & 1) * NUM_CLASSES
                for cg in range(0, NUM_CLASSES, 8):
                    vals = [
                        plsc.load_gather(gbuf, [rows, par + (cg + t)])
                        for t in range(8)
                    ]
                    for t in range(8):
                        obuf[cg + t, pl.ds(16 * k, 16)] = vals[t]

        def out_slice(j):
            g = gbase + j
            return out_hbm.at[g // nbc, :, pl.ds((g % nbc) * _CB, _CB)]

        def half_step(i, j, gbuf, obuf, gsem, ssem):
            pltpu.make_async_copy(p_hbm.at[hlf_v.at[j]], gbuf, gsem).wait()

            @pl.when(i > 0)
            def _():
                pltpu.make_async_copy(obuf, out_slice(j - 2), ssem).wait()

            compact(j, gbuf, obuf)

            @pl.when(j + 2 < cpw)
            def _():
                pltpu.async_copy(p_hbm.at[hlf_v.at[j + 2]], gbuf, gsem)

            pltpu.async_copy(obuf, out_slice(j), ssem)

        def body(i, carry):
            j = 2 * i
            half_step(i, j, gbuf0, obuf0, gsem0, ssem0)
            half_step(i, j + 1, gbuf1, obuf1, gsem1, ssem1)
            return carry

        lax.fori_loop(0, cpw // 2, body, 0)
        pltpu.make_async_copy(obuf0, out_slice(cpw - 2), ssem0).wait()
        pltpu.make_async_copy(obuf1, out_slice(cpw - 1), ssem1).wait()

    return gather_k


def kernel(token_ids, emb_table, W, b):
    B, L = token_ids.shape
    info = plsc.get_sparse_core_info()
    nw = info.num_cores * info.num_subcores

    emb2 = emb_table.reshape(VOCAB // 2, 2 * EMB_DIM)
    w_blk = (
        jnp.zeros((2 * EMB_DIM, 2 * NUM_CLASSES), jnp.float32)
        .at[:EMB_DIM, :NUM_CLASSES].set(W)
        .at[EMB_DIM:, NUM_CLASSES:].set(W)
    )
    b2 = jnp.concatenate([b, b]).reshape(1, 2 * NUM_CLASSES)
    proj = _project(emb2, w_blk, b2)

    ids2 = token_ids.T.reshape((B // _CB) * L, _CB).astype(jnp.int32)
    out3 = _make_gather(B, L, nw)(ids2, proj)
    return out3.transpose(2, 0, 1)
